# baseline (speedup 1.0000x reference)
import functools

import jax
import jax.numpy as jnp
from jax import lax
from jax.experimental import pallas as pl
from jax.experimental.pallas import tpu as pltpu
from jax.experimental.pallas import tpu_sc as plsc

_B = 2048
_BLK = 256
_K = 512
_D = 256

_NC, _NS = 2, 16
_NW = _NC * _NS
_BPW = _B // _NW


def _pick_block(x_ref, emb_ref, embT_ref, idx_ref):
    x = x_ref[...]
    emb = emb_ref[...]
    embT = embT_ref[...]

    esq = jnp.sum(emb * emb, axis=0)
    dots = jax.lax.dot_general(
        x, emb, (((1,), (0,)), ((), ())),
        precision=jax.lax.Precision.HIGHEST,
        preferred_element_type=jnp.float32)
    s = esq[None, :] - 2.0 * dots

    kidx = jax.lax.broadcasted_iota(jnp.int32, s.shape, 1)
    m1 = jnp.min(s, axis=1, keepdims=True)
    i1 = jnp.min(jnp.where(s == m1, kidx, _K), axis=1)
    s2 = jnp.where(kidx == i1[:, None], jnp.inf, s)
    m2 = jnp.min(s2, axis=1, keepdims=True)
    i2 = jnp.min(jnp.where(s2 == m2, kidx, _K), axis=1)

    oh1 = (kidx == i1[:, None]).astype(jnp.float32)
    oh2 = (kidx == i2[:, None]).astype(jnp.float32)
    e1 = jax.lax.dot_general(
        oh1, embT, (((1,), (0,)), ((), ())),
        precision=jax.lax.Precision.HIGHEST,
        preferred_element_type=jnp.float32)
    e2 = jax.lax.dot_general(
        oh2, embT, (((1,), (0,)), ((), ())),
        precision=jax.lax.Precision.HIGHEST,
        preferred_element_type=jnp.float32)

    d1 = jnp.sum((x - e1) ** 2, axis=1)
    d2 = jnp.sum((x - e2) ** 2, axis=1)
    pick1 = (d1 < d2) | ((d1 == d2) & (i1 < i2))
    idx_ref[0, 0, :] = jnp.where(pick1, i1, i2)


def _tc_pick(x, weight, weight_t):
    nblk = _B // _BLK
    idx = pl.pallas_call(
        _pick_block,
        grid=(nblk,),
        in_specs=[
            pl.BlockSpec((_BLK, _D), lambda i: (i, 0)),
            pl.BlockSpec((_D, _K), lambda i: (0, 0)),
            pl.BlockSpec((_K, _D), lambda i: (0, 0)),
        ],
        out_specs=pl.BlockSpec((1, 1, _BLK), lambda i: (i, 0, 0)),
        out_shape=jax.ShapeDtypeStruct((nblk, 1, _BLK), jnp.int32),
    )(x, weight, weight_t)
    return idx.reshape(_B)


def _sc_gather_body(table_hbm, idx_hbm, out_hbm, idx_v, rows_v, sem):
    wid = lax.axis_index("s") * _NC + lax.axis_index("c")
    base = wid * _BPW
    pltpu.sync_copy(idx_hbm.at[pl.ds(base, _BPW)], idx_v)
    pltpu.async_copy(table_hbm.at[idx_v], rows_v, sem).wait()
    pltpu.sync_copy(rows_v, out_hbm.at[pl.ds(base, _BPW)])


def _sc_gather(table, idx):
    k = functools.partial(
        pl.kernel,
        mesh=plsc.VectorSubcoreMesh(core_axis_name="c", subcore_axis_name="s"),
        out_type=jax.ShapeDtypeStruct((_B, _D), jnp.float32),
        scratch_types=[
            pltpu.VMEM((_BPW,), jnp.int32),
            pltpu.VMEM((_BPW, _D), jnp.float32),
            pltpu.SemaphoreType.DMA,
        ],
    )(_sc_gather_body)
    return k(table, idx)


@jax.jit
def kernel(x, weight):
    weight_t = weight.T
    idx = _tc_pick(x, weight, weight_t)
    return _sc_gather(weight_t, idx)
